# baseline (speedup 1.0000x reference)
import functools

import jax
import jax.numpy as jnp
from jax import lax
from jax.experimental import pallas as pl
from jax.experimental.pallas import tpu as pltpu
from jax.experimental.pallas import tpu_sc as plsc

_B, _T, _V = 8, 2048, 32
_NG = 16
_NQ = 2 * _NG
_L = 16
_NW = 32
_PPW = _B * _V // _NW
_VPW = _V // (_NW // _B)


def _sc_glimpse(vals_t, time, masks_t, l_b):
    mesh = plsc.VectorSubcoreMesh(core_axis_name="c", subcore_axis_name="s")

    @functools.partial(
        pl.kernel,
        out_type=jax.ShapeDtypeStruct((_B, _V * _NQ), jnp.float32),
        mesh=mesh,
        compiler_params=pltpu.CompilerParams(needs_layout_passes=False),
        scratch_types=[
            pltpu.VMEM((_T,), jnp.float32),
            pltpu.VMEM((_T,), jnp.float32),
            pltpu.VMEM((_T,), jnp.int32),
            pltpu.VMEM((_T + _L,), jnp.float32),
            pltpu.VMEM((_T + _L,), jnp.float32),
            pltpu.VMEM((_L,), jnp.float32),
            pltpu.VMEM((_VPW * _NQ,), jnp.float32),
        ],
    )
    def body(vals_hbm, time_hbm, masks_hbm, l_hbm, out_hbm,
             t_v, x_v, m_v, obs_t, obs_v, l_v, o_v):
        wid = lax.axis_index("s") * 2 + lax.axis_index("c")
        b = wid // (_NW // _B)
        v0 = (wid % (_NW // _B)) * _VPW
        pltpu.sync_copy(time_hbm.at[b], t_v)
        pltpu.sync_copy(l_hbm.at[b], l_v)
        lvec = l_v[...]
        iotaf = lax.iota(jnp.int32, _L).astype(jnp.float32)
        lin0 = iotaf * jnp.float32(0.1 / 15.0) + jnp.float32(-0.05)
        lin1 = iotaf * jnp.float32(0.5 / 15.0) + jnp.float32(-0.25)

        def pair_body(j, carry):
            v = v0 + j
            pltpu.sync_copy(vals_hbm.at[b, v], x_v)
            pltpu.sync_copy(masks_hbm.at[b, v], m_v)

            def comp_body(i, cntv):
                off = i * _L
                mi = m_v[pl.ds(off, _L)]
                mm = mi != 0
                dest = cntv + jnp.cumsum(mi) - 1
                plsc.store_scatter(obs_t, [dest], t_v[pl.ds(off, _L)], mask=mm)
                plsc.store_scatter(obs_v, [dest], x_v[pl.ds(off, _L)], mask=mm)
                return cntv + plsc.all_reduce_population_count(mm)

            nvec = lax.fori_loop(0, _T // _L, comp_body,
                                 jnp.zeros((_L,), jnp.int32))
            last_idx = jnp.maximum(nvec - 1, 0)
            last_t = plsc.load_gather(obs_t, [last_idx])
            last_v = plsc.load_gather(obs_v, [last_idx])
            idx_hi = jnp.maximum(nvec - 2, 0)

            def interp(lin):
                r = (lin + lvec) * last_t
                cnt = jnp.zeros((_L,), jnp.int32)
                step = _T
                while step >= 1:
                    cand = cnt + step
                    gidx = jnp.minimum(cand - 1, last_idx)
                    tv = plsc.load_gather(obs_t, [gidx])
                    ok = (cand <= nvec) & (tv <= r)
                    cnt = jnp.where(ok, cand, cnt)
                    step //= 2
                idx0 = jnp.clip(cnt - 1, 0, idx_hi)
                idx1 = idx0 + 1
                x0 = plsc.load_gather(obs_t, [idx0])
                x1 = plsc.load_gather(obs_t, [idx1])
                y0 = plsc.load_gather(obs_v, [idx0])
                y1 = plsc.load_gather(obs_v, [idx1])
                y = y0 + (r - x0) / (x1 - x0) * (y1 - y0)
                y = jnp.where(r < x0, y0, y)
                y = jnp.where(r > last_t, last_v, y)
                y = jnp.where(nvec == 1, last_v, y)
                y = jnp.where(nvec == 0, jnp.zeros((_L,), jnp.float32), y)
                return y

            o_v[pl.ds(j * _NQ, _L)] = interp(lin0)
            o_v[pl.ds(j * _NQ + _NG, _L)] = interp(lin1)
            return carry

        lax.fori_loop(0, _VPW, pair_body, jnp.int32(0))
        pltpu.sync_copy(o_v, out_hbm.at[b, pl.ds(v0 * _NQ, _VPW * _NQ)])

    return body(vals_t, time, masks_t, l_b)


def _tc_fc(g, l_t, wgt, wl, brow, nhid):
    def fc_body(g_ref, l_ref, w_ref, wl_ref, b_ref, o_ref):
        o_ref[...] = (
            jnp.dot(g_ref[...], w_ref[...], preferred_element_type=jnp.float32)
            + l_ref[...] * wl_ref[...]
            + b_ref[...]
        )

    return pl.pallas_call(
        fc_body,
        out_shape=jax.ShapeDtypeStruct((_B, nhid), jnp.float32),
    )(g, l_t, wgt, wl, brow)


def kernel(vals, time, masks, lengths, l_t, W, b):
    del lengths
    nhid = W.shape[0]
    vals_t = jnp.transpose(vals, (0, 2, 1))
    masks_t = jnp.transpose(masks, (0, 2, 1)).astype(jnp.int32)
    l_b = jnp.broadcast_to(l_t, (_B, _L))
    g = _sc_glimpse(vals_t, time, masks_t, l_b)
    wgt = W[:, :-1].T
    wl = W[:, -1].reshape(1, nhid)
    grep = _tc_fc(g, l_t, wgt, wl, b.reshape(1, nhid), nhid)
    return grep, g[:, g.shape[1] // 2]


# speedup vs baseline: 1.0085x; 1.0085x over previous
import functools

import jax
import jax.numpy as jnp
from jax import lax
from jax.experimental import pallas as pl
from jax.experimental.pallas import tpu as pltpu
from jax.experimental.pallas import tpu_sc as plsc

_B, _T, _V = 8, 2048, 32
_NG = 16
_NQ = 2 * _NG
_L = 16
_NW = 32
_VPW = _V // (_NW // _B)
_SEG = _T // _L


def _sc_glimpse(vals_nt, time, l_b):
    mesh = plsc.VectorSubcoreMesh(core_axis_name="c", subcore_axis_name="s")

    @functools.partial(
        pl.kernel,
        out_type=jax.ShapeDtypeStruct((_B, _V * _NQ), jnp.float32),
        mesh=mesh,
        compiler_params=pltpu.CompilerParams(needs_layout_passes=False),
        scratch_types=[
            pltpu.VMEM((_T,), jnp.float32),
            pltpu.VMEM((2 * _T,), jnp.float32),
            pltpu.VMEM((_T + _L,), jnp.float32),
            pltpu.VMEM((_T + _L,), jnp.float32),
            pltpu.VMEM((_L,), jnp.float32),
            pltpu.VMEM((_VPW * _NQ,), jnp.float32),
            pltpu.SemaphoreType.DMA,
            pltpu.SemaphoreType.DMA,
        ],
    )
    def body(vals_hbm, time_hbm, l_hbm, out_hbm,
             t_v, x2, obs_t, obs_v, l_v, o_v, semx0, semx1):
        wid = lax.axis_index("s") * 2 + lax.axis_index("c")
        b = wid // (_NW // _B)
        v0 = (wid % (_NW // _B)) * _VPW
        semx = (semx0, semx1)

        def start_fetch(j):
            p = j & 1
            return pltpu.async_copy(vals_hbm.at[b, v0 + j],
                                    x2.at[pl.ds(p * _T, _T)], semx[p])

        cp = start_fetch(0)
        pltpu.sync_copy(time_hbm.at[b], t_v)
        pltpu.sync_copy(l_hbm.at[b], l_v)
        lvec = l_v[...]
        iota = lax.iota(jnp.int32, _L)
        iotaf = iota.astype(jnp.float32)
        lin0 = iotaf * jnp.float32(0.1 / 15.0) + jnp.float32(-0.05)
        lin1 = iotaf * jnp.float32(0.5 / 15.0) + jnp.float32(-0.25)
        lane_t = iota * _SEG

        for j in range(_VPW):
            p = j & 1
            nxt = start_fetch(j + 1) if j + 1 < _VPW else None
            cp.wait()
            cp = nxt
            lane_x = lane_t + p * _T

            def cnt_body(i, c, lane_x=lane_x):
                xi = plsc.load_gather(x2, [lane_x + i])
                return c + (xi == xi).astype(jnp.int32)

            c = plsc.parallel_loop(0, _SEG, 1, unroll=4,
                                   carry=jnp.zeros((_L,), jnp.int32))(cnt_body)
            offs = jnp.cumsum(c) - c
            nvec = jnp.full((_L,), jnp.sum(c), jnp.int32)

            def sc_body(i, dest, lane_x=lane_x):
                xi = plsc.load_gather(x2, [lane_x + i])
                ti = plsc.load_gather(t_v, [lane_t + i])
                mm = xi == xi
                plsc.store_scatter(obs_t, [dest], ti, mask=mm)
                plsc.store_scatter(obs_v, [dest], xi, mask=mm)
                return dest + mm.astype(jnp.int32)

            plsc.parallel_loop(0, _SEG, 1, unroll=4, carry=offs)(sc_body)

            last_idx = jnp.maximum(nvec - 1, 0)
            last_t = plsc.load_gather(obs_t, [last_idx])
            last_v = plsc.load_gather(obs_v, [last_idx])
            idx_hi = jnp.maximum(nvec - 2, 0)

            def interp(lin):
                r = (lin + lvec) * last_t
                cnt = jnp.zeros((_L,), jnp.int32)
                step = _T // 2
                while step >= 1:
                    cand = cnt + step
                    gidx = jnp.minimum(cand - 1, last_idx)
                    tv = plsc.load_gather(obs_t, [gidx])
                    ok = (cand <= nvec) & (tv <= r)
                    cnt = jnp.where(ok, cand, cnt)
                    step //= 2
                idx0 = jnp.clip(cnt - 1, 0, idx_hi)
                idx1 = idx0 + 1
                x0 = plsc.load_gather(obs_t, [idx0])
                x1 = plsc.load_gather(obs_t, [idx1])
                y0 = plsc.load_gather(obs_v, [idx0])
                y1 = plsc.load_gather(obs_v, [idx1])
                y = y0 + (r - x0) / (x1 - x0) * (y1 - y0)
                y = jnp.where(r < x0, y0, y)
                y = jnp.where(r > last_t, last_v, y)
                y = jnp.where(nvec == 1, last_v, y)
                y = jnp.where(nvec == 0, jnp.zeros((_L,), jnp.float32), y)
                return y

            o_v[pl.ds(j * _NQ, _L)] = interp(lin0)
            o_v[pl.ds(j * _NQ + _NG, _L)] = interp(lin1)

        pltpu.sync_copy(o_v, out_hbm.at[b, pl.ds(v0 * _NQ, _VPW * _NQ)])

    return body(vals_nt, time, l_b)


def _tc_fc(g, l_t, wgt, wl, brow, nhid):
    def fc_body(g_ref, l_ref, w_ref, wl_ref, b_ref, o_ref):
        o_ref[...] = (
            jnp.dot(g_ref[...], w_ref[...], preferred_element_type=jnp.float32)
            + l_ref[...] * wl_ref[...]
            + b_ref[...]
        )

    return pl.pallas_call(
        fc_body,
        out_shape=jax.ShapeDtypeStruct((_B, nhid), jnp.float32),
    )(g, l_t, wgt, wl, brow)


def kernel(vals, time, masks, lengths, l_t, W, b):
    del lengths
    nhid = W.shape[0]
    vals_nt = jnp.transpose(
        jnp.where(masks, vals, jnp.float32(jnp.nan)), (0, 2, 1))
    l_b = jnp.broadcast_to(l_t, (_B, _L))
    g = _sc_glimpse(vals_nt, time, l_b)
    wgt = W[:, :-1].T
    wl = W[:, -1].reshape(1, nhid)
    grep = _tc_fc(g, l_t, wgt, wl, b.reshape(1, nhid), nhid)
    return grep, g[:, g.shape[1] // 2]


# speedup vs baseline: 1.5630x; 1.5498x over previous
import functools

import jax
import jax.numpy as jnp
from jax import lax
from jax.experimental import pallas as pl
from jax.experimental.pallas import tpu as pltpu
from jax.experimental.pallas import tpu_sc as plsc

_B, _T, _V = 8, 2048, 32
_NG = 16
_NQ = 2 * _NG
_L = 16
_NW = 32
_PPW = _B * _V // _NW
_VPW = _V // (_NW // _B)


def _sc_glimpse(vals_nt, time, l_b):
    mesh = plsc.VectorSubcoreMesh(core_axis_name="c", subcore_axis_name="s")

    @functools.partial(
        pl.kernel,
        out_type=jax.ShapeDtypeStruct((_B, _V * _NQ), jnp.float32),
        mesh=mesh,
        compiler_params=pltpu.CompilerParams(needs_layout_passes=False),
        scratch_types=[
            pltpu.VMEM((_T,), jnp.float32),
            pltpu.VMEM((2 * _T,), jnp.float32),
            pltpu.VMEM((_T + _L,), jnp.float32),
            pltpu.VMEM((_T + _L,), jnp.float32),
            pltpu.VMEM((_L,), jnp.float32),
            pltpu.VMEM((_VPW * _NQ,), jnp.float32),
            pltpu.SemaphoreType.DMA,
            pltpu.SemaphoreType.DMA,
        ],
    )
    def body(vals_hbm, time_hbm, l_hbm, out_hbm,
             t_v, x2, obs_t, obs_v, l_v, o_v, semx0, semx1):
        wid = lax.axis_index("s") * 2 + lax.axis_index("c")
        b = wid // (_NW // _B)
        v0 = (wid % (_NW // _B)) * _VPW
        semx = (semx0, semx1)

        def start_fetch(j):
            p = j & 1
            return pltpu.async_copy(vals_hbm.at[b, v0 + j],
                                    x2.at[pl.ds(p * _T, _T)], semx[p])

        cp = start_fetch(0)
        pltpu.sync_copy(time_hbm.at[b], t_v)
        pltpu.sync_copy(l_hbm.at[b], l_v)
        lvec = l_v[...]
        iotaf = lax.iota(jnp.int32, _L).astype(jnp.float32)
        lin0 = iotaf * jnp.float32(0.1 / 15.0) + jnp.float32(-0.05)
        lin1 = iotaf * jnp.float32(0.5 / 15.0) + jnp.float32(-0.25)

        for j in range(_VPW):
            p = j & 1
            nxt = start_fetch(j + 1) if j + 1 < _VPW else None
            cp.wait()
            cp = nxt
            base = p * _T

            def comp_body(i, cntv, base=base):
                off = i * _L
                xi = x2[pl.ds(base + off, _L)]
                mm = xi == xi
                dest = cntv + jnp.cumsum(mm.astype(jnp.int32)) - 1
                plsc.store_scatter(obs_t, [dest], t_v[pl.ds(off, _L)], mask=mm)
                plsc.store_scatter(obs_v, [dest], xi, mask=mm)
                return cntv + plsc.all_reduce_population_count(mm)

            nvec = plsc.parallel_loop(0, _T // _L, 1, unroll=4,
                                      carry=jnp.zeros((_L,), jnp.int32))(comp_body)
            last_idx = jnp.maximum(nvec - 1, 0)
            last_t = plsc.load_gather(obs_t, [last_idx])
            last_v = plsc.load_gather(obs_v, [last_idx])
            idx_hi = jnp.maximum(nvec - 2, 0)

            def interp(lin):
                r = (lin + lvec) * last_t
                cnt = jnp.zeros((_L,), jnp.int32)
                step = _T // 2
                while step >= 1:
                    cand = cnt + step
                    gidx = jnp.minimum(cand - 1, last_idx)
                    tv = plsc.load_gather(obs_t, [gidx])
                    ok = (cand <= nvec) & (tv <= r)
                    cnt = jnp.where(ok, cand, cnt)
                    step //= 2
                idx0 = jnp.clip(cnt - 1, 0, idx_hi)
                idx1 = idx0 + 1
                x0 = plsc.load_gather(obs_t, [idx0])
                x1 = plsc.load_gather(obs_t, [idx1])
                y0 = plsc.load_gather(obs_v, [idx0])
                y1 = plsc.load_gather(obs_v, [idx1])
                y = y0 + (r - x0) / (x1 - x0) * (y1 - y0)
                y = jnp.where(r < x0, y0, y)
                y = jnp.where(r > last_t, last_v, y)
                y = jnp.where(nvec == 1, last_v, y)
                y = jnp.where(nvec == 0, jnp.zeros((_L,), jnp.float32), y)
                return y

            o_v[pl.ds(j * _NQ, _L)] = interp(lin0)
            o_v[pl.ds(j * _NQ + _NG, _L)] = interp(lin1)

        pltpu.sync_copy(o_v, out_hbm.at[b, pl.ds(v0 * _NQ, _VPW * _NQ)])

    return body(vals_nt, time, l_b)


def _tc_fc(g, l_t, wgt, wl, brow, nhid):
    def fc_body(g_ref, l_ref, w_ref, wl_ref, b_ref, o_ref):
        o_ref[...] = (
            jnp.dot(g_ref[...], w_ref[...], preferred_element_type=jnp.float32)
            + l_ref[...] * wl_ref[...]
            + b_ref[...]
        )

    return pl.pallas_call(
        fc_body,
        out_shape=jax.ShapeDtypeStruct((_B, nhid), jnp.float32),
    )(g, l_t, wgt, wl, brow)


def kernel(vals, time, masks, lengths, l_t, W, b):
    del lengths
    nhid = W.shape[0]
    vals_nt = jnp.transpose(
        jnp.where(masks, vals, jnp.float32(jnp.nan)), (0, 2, 1))
    l_b = jnp.broadcast_to(l_t, (_B, _L))
    g = _sc_glimpse(vals_nt, time, l_b)
    wgt = W[:, :-1].T
    wl = W[:, -1].reshape(1, nhid)
    grep = _tc_fc(g, l_t, wgt, wl, b.reshape(1, nhid), nhid)
    return grep, g[:, g.shape[1] // 2]
